# baseline (speedup 1.0000x reference)
import jax
import jax.numpy as jnp
from jax import lax
from jax.experimental import pallas as pl
from jax.experimental.pallas import tpu as pltpu

T_CHUNK = 128
UNROLL = 32

_DN_T = (((1,), (1,)), ((), ()))


def _lstm_body(sv_ref, wih_ref, whh_ref, bias_ref, out_ref, xg_ref, h_ref, c_ref):
    i = pl.program_id(0)
    nb = sv_ref.shape[0]
    hdim = h_ref.shape[1]

    @pl.when(i == 0)
    def _init():
        h_ref[...] = jnp.zeros_like(h_ref)
        c_ref[...] = jnp.zeros_like(c_ref)

    sv_tm = jnp.swapaxes(sv_ref[...], 0, 1).reshape(T_CHUNK * nb, sv_ref.shape[2])
    xg_ref[...] = (
        lax.dot_general(sv_tm, wih_ref[...], _DN_T, preferred_element_type=jnp.float32)
        + bias_ref[...]
    )

    whh = whh_ref[...]

    def one_step(t, h, c):
        hb = h.astype(jnp.bfloat16)
        xg = xg_ref[pl.ds(t * nb, nb), :]

        def gate(k):
            return xg[:, k * hdim : (k + 1) * hdim] + jnp.dot(
                hb,
                whh[:, k * hdim : (k + 1) * hdim],
                preferred_element_type=jnp.float32,
            )

        gi = jax.nn.sigmoid(gate(0))
        gf = jax.nn.sigmoid(gate(1))
        gg = jnp.tanh(gate(2))
        c_new = gf * c + gi * gg
        go = jax.nn.sigmoid(gate(3))
        h_new = go * jnp.tanh(c_new)
        return h_new, c_new

    def step(s, carry):
        h, c = carry
        for k in range(UNROLL):
            h, c = one_step(s * UNROLL + k, h, c)
        return h, c

    h, c = lax.fori_loop(0, T_CHUNK // UNROLL, step, (h_ref[...], c_ref[...]))
    h_ref[...] = h
    c_ref[...] = c

    @pl.when(i == pl.num_programs(0) - 1)
    def _emit():
        out_ref[...] = h


def kernel(sv, W_ih, W_hh, b_ih, b_hh):
    b, l, d = sv.shape
    h4 = W_ih.shape[0]
    hdim = W_hh.shape[1]
    nchunk = l // T_CHUNK

    whh_bf = W_hh.T.astype(jnp.bfloat16)
    bias = (b_ih + b_hh).reshape(1, h4)

    return pl.pallas_call(
        _lstm_body,
        grid=(nchunk,),
        in_specs=[
            pl.BlockSpec((b, T_CHUNK, d), lambda i: (0, i, 0)),
            pl.BlockSpec((h4, d), lambda i: (0, 0)),
            pl.BlockSpec((hdim, h4), lambda i: (0, 0)),
            pl.BlockSpec((1, h4), lambda i: (0, 0)),
        ],
        out_specs=pl.BlockSpec((b, hdim), lambda i: (0, 0)),
        out_shape=jax.ShapeDtypeStruct((b, hdim), jnp.float32),
        scratch_shapes=[
            pltpu.VMEM((T_CHUNK * b, h4), jnp.float32),
            pltpu.VMEM((b, hdim), jnp.float32),
            pltpu.VMEM((b, hdim), jnp.float32),
        ],
    )(sv, W_ih, whh_bf, bias)


# speedup vs baseline: 1.0151x; 1.0151x over previous
import jax
import jax.numpy as jnp
from jax import lax
from jax.experimental import pallas as pl
from jax.experimental.pallas import tpu as pltpu

T_CHUNK = 128
UNROLL = 16

_DN_T = (((1,), (1,)), ((), ()))


def _lstm_body(
    sv_ref, wih_ref, whh_ref, bias_ref, out_ref, xg_ref, h_ref, c_ref, whhb_ref
):
    i = pl.program_id(0)
    nb = sv_ref.shape[0]
    hdim = h_ref.shape[1]

    @pl.when(i == 0)
    def _init():
        h_ref[...] = jnp.zeros_like(h_ref)
        c_ref[...] = jnp.zeros_like(c_ref)
        whhb_ref[...] = jnp.swapaxes(whh_ref[...], 0, 1).astype(jnp.bfloat16)

    sv_tm = jnp.swapaxes(sv_ref[...], 0, 1).reshape(T_CHUNK * nb, sv_ref.shape[2])
    xg_ref[...] = (
        lax.dot_general(sv_tm, wih_ref[...], _DN_T, preferred_element_type=jnp.float32)
        + bias_ref[...]
    )

    whh = whhb_ref[...]

    def one_step(t, h, c):
        hb = h.astype(jnp.bfloat16)
        xg = xg_ref[pl.ds(t * nb, nb), :]

        def gate(k):
            return xg[:, k * hdim : (k + 1) * hdim] + jnp.dot(
                hb,
                whh[:, k * hdim : (k + 1) * hdim],
                preferred_element_type=jnp.float32,
            )

        gi = jax.nn.sigmoid(gate(0))
        gf = jax.nn.sigmoid(gate(1))
        gg = jnp.tanh(gate(2))
        c_new = gf * c + gi * gg
        go = jax.nn.sigmoid(gate(3))
        h_new = go * jnp.tanh(c_new)
        return h_new, c_new

    def step(s, carry):
        h, c = carry
        for k in range(UNROLL):
            h, c = one_step(s * UNROLL + k, h, c)
        return h, c

    h, c = lax.fori_loop(0, T_CHUNK // UNROLL, step, (h_ref[...], c_ref[...]))
    h_ref[...] = h
    c_ref[...] = c

    @pl.when(i == pl.num_programs(0) - 1)
    def _emit():
        out_ref[...] = h


def kernel(sv, W_ih, W_hh, b_ih, b_hh):
    b, l, d = sv.shape
    h4 = W_ih.shape[0]
    hdim = W_hh.shape[1]
    nchunk = l // T_CHUNK

    bias = (b_ih + b_hh).reshape(1, h4)

    return pl.pallas_call(
        _lstm_body,
        grid=(nchunk,),
        in_specs=[
            pl.BlockSpec((b, T_CHUNK, d), lambda i: (0, i, 0)),
            pl.BlockSpec((h4, d), lambda i: (0, 0)),
            pl.BlockSpec((h4, hdim), lambda i: (0, 0)),
            pl.BlockSpec((1, h4), lambda i: (0, 0)),
        ],
        out_specs=pl.BlockSpec((b, hdim), lambda i: (0, 0)),
        out_shape=jax.ShapeDtypeStruct((b, hdim), jnp.float32),
        scratch_shapes=[
            pltpu.VMEM((T_CHUNK * b, h4), jnp.float32),
            pltpu.VMEM((b, hdim), jnp.float32),
            pltpu.VMEM((b, hdim), jnp.float32),
            pltpu.VMEM((hdim, h4), jnp.bfloat16),
        ],
    )(sv, W_ih, W_hh, bias)
